# baseline (speedup 1.0000x reference)
import functools

import jax
import jax.numpy as jnp
from jax import lax
from jax.experimental import pallas as pl
from jax.experimental.pallas import tpu as pltpu
from jax.experimental.pallas import tpu_sc as plsc

F32 = jnp.float32
I32 = jnp.int32

N_C = 10000
N_F = 20000
NFP = 20480
NCP = 10240
C_IN = 128
C_MID = 64
C_OUT = 128
E_C = 160000
E_F = 320000

BLK = 256
NBLK = NFP // BLK

SELU_ALPHA = 1.6732632423543772
SELU_SCALE = 1.0507009873554805


def _selu(v):
    return SELU_SCALE * jnp.where(v > 0, v, SELU_ALPHA * (jnp.exp(v) - 1.0))


KBLK = 512
KNBLK = NFP // KBLK
_BIG_I = NCP
_INF = 3e38


def _knn_body(ps_ref, pd_ref, idx_ref, wn_ref, d2_ref):
    acc = None
    for d in range(3):
        ps_d = ps_ref[:, d:d + 1]
        pd_d = pd_ref[0, d:d + 1, :]
        t = ps_d - pd_d
        t = t * t
        acc = t if acc is None else acc + t
    d2_ref[...] = acc
    iota = lax.broadcasted_iota(I32, (NCP, KBLK), 0)
    t = d2_ref[...]
    m1 = jnp.min(t, axis=0, keepdims=True)
    a1 = jnp.min(jnp.where(t == m1, iota, _BIG_I), axis=0, keepdims=True)
    not1 = iota != a1
    m2 = jnp.min(jnp.where(not1, t, _INF), axis=0, keepdims=True)
    a2 = jnp.min(jnp.where((t == m2) & not1, iota, _BIG_I), axis=0,
                 keepdims=True)
    not12 = not1 & (iota != a2)
    m3 = jnp.min(jnp.where(not12, t, _INF), axis=0, keepdims=True)
    a3 = jnp.min(jnp.where((t == m3) & not12, iota, _BIG_I), axis=0,
                 keepdims=True)
    w1 = 1.0 / (jnp.maximum(m1, 0.0) + 1e-8)
    w2 = 1.0 / (jnp.maximum(m2, 0.0) + 1e-8)
    w3 = 1.0 / (jnp.maximum(m3, 0.0) + 1e-8)
    s = w1 + w2 + w3
    for k, (a, w) in enumerate(((a1, w1), (a2, w2), (a3, w3))):
        idx_ref[0, k:k + 1, :] = a
        wn_ref[0, k:k + 1, :] = w / s


def _knn(ps_pad, pd_blocks):
    return pl.pallas_call(
        _knn_body,
        grid=(KNBLK,),
        in_specs=[
            pl.BlockSpec((NCP, 8), lambda i: (0, 0)),
            pl.BlockSpec((1, 8, KBLK), lambda i: (i, 0, 0)),
        ],
        out_specs=[
            pl.BlockSpec((1, 3, KBLK), lambda i: (i, 0, 0)),
            pl.BlockSpec((1, 3, KBLK), lambda i: (i, 0, 0)),
        ],
        out_shape=[
            jax.ShapeDtypeStruct((KNBLK, 3, KBLK), I32),
            jax.ShapeDtypeStruct((KNBLK, 3, KBLK), F32),
        ],
        scratch_shapes=[pltpu.VMEM((NCP, KBLK), F32)],
    )(ps_pad, pd_blocks)


M1_BLK = 1000


def _m1_body(x_ref, ws_ref, wm_ref, as_ref, am_ref):
    xb = x_ref[...]
    as_ref[...] = jnp.dot(xb, ws_ref[...], preferred_element_type=F32)
    am_ref[...] = jnp.dot(xb, wm_ref[...], preferred_element_type=F32)


def _m1(x, ws, wm):
    return pl.pallas_call(
        _m1_body,
        grid=(N_C // M1_BLK,),
        in_specs=[
            pl.BlockSpec((M1_BLK, C_IN), lambda i: (i, 0)),
            pl.BlockSpec((C_IN, C_MID), lambda i: (0, 0)),
            pl.BlockSpec((C_IN, C_MID), lambda i: (0, 0)),
        ],
        out_specs=[
            pl.BlockSpec((M1_BLK, C_MID), lambda i: (i, 0)),
            pl.BlockSpec((M1_BLK, C_MID), lambda i: (i, 0)),
        ],
        out_shape=[jax.ShapeDtypeStruct((N_C, C_MID), F32)] * 2,
    )(x, ws, wm)


def _p_body(as_ref, agg_ref, b1_ref, x_ref, w2s_ref, w2m_ref, wss_ref,
            wsm_ref, p_ref):
    h = _selu(as_ref[...] + agg_ref[0] + agg_ref[1] + b1_ref[...])
    xb = x_ref[...]
    p_ref[...] = jnp.concatenate(
        [
            jnp.dot(h, w2s_ref[...], preferred_element_type=F32),
            jnp.dot(xb, wss_ref[...], preferred_element_type=F32),
            jnp.dot(h, w2m_ref[...], preferred_element_type=F32),
            jnp.dot(xb, wsm_ref[...], preferred_element_type=F32),
        ],
        axis=1,
    )


def _p_kernel(a_self, agg_c, b1, x, w2s, w2m, wss, wsm):
    return pl.pallas_call(
        _p_body,
        grid=(N_C // M1_BLK,),
        in_specs=[
            pl.BlockSpec((M1_BLK, C_MID), lambda i: (i, 0)),
            pl.BlockSpec((2, M1_BLK, C_MID), lambda i: (0, i, 0)),
            pl.BlockSpec((1, C_MID), lambda i: (0, 0)),
            pl.BlockSpec((M1_BLK, C_IN), lambda i: (i, 0)),
            pl.BlockSpec((C_MID, C_OUT), lambda i: (0, 0)),
            pl.BlockSpec((C_MID, C_OUT), lambda i: (0, 0)),
            pl.BlockSpec((C_IN, C_OUT), lambda i: (0, 0)),
            pl.BlockSpec((C_IN, C_OUT), lambda i: (0, 0)),
        ],
        out_specs=pl.BlockSpec((M1_BLK, 4 * C_OUT), lambda i: (i, 0)),
        out_shape=jax.ShapeDtypeStruct((N_C, 4 * C_OUT), F32),
    )(a_self, agg_c, b1, x, w2s, w2m, wss, wsm)


def _w_body(g_ref, wn_ref, pself_ref, pmsg_ref):
    p = None
    for k in range(3):
        wk = wn_ref[:, k:k + 1]
        t = g_ref[k] * wk
        p = t if p is None else p + t
    pself_ref[...] = p[:, :2 * C_OUT]
    for g in range(4):
        pmsg_ref[g] = p[:, 2 * C_OUT + C_MID * g: 2 * C_OUT + C_MID * (g + 1)]


def _w_kernel(gtab, wn8):
    return pl.pallas_call(
        _w_body,
        grid=(NBLK,),
        in_specs=[
            pl.BlockSpec((3, BLK, 4 * C_OUT), lambda i: (0, i, 0)),
            pl.BlockSpec((BLK, 8), lambda i: (i, 0)),
        ],
        out_specs=[
            pl.BlockSpec((BLK, 2 * C_OUT), lambda i: (i, 0)),
            pl.BlockSpec((4, BLK, C_MID), lambda i: (0, i, 0)),
        ],
        out_shape=[
            jax.ShapeDtypeStruct((NFP, 2 * C_OUT), F32),
            jax.ShapeDtypeStruct((4, NFP, C_MID), F32),
        ],
    )(gtab, wn8)


def _f1_body(ps_ref, ag_ref, b2_ref, bs_ref, o_ref, st_ref):
    i = pl.program_id(0)
    ms = ps_ref[:, :C_OUT]
    ss = ps_ref[:, C_OUT:]
    am = jnp.concatenate([ag_ref[0], ag_ref[1]], axis=1)
    ak = jnp.concatenate([ag_ref[2], ag_ref[3]], axis=1)
    o = _selu(ms + am + b2_ref[...]) + _selu(ss + ak + bs_ref[...])
    o_ref[...] = o
    rows = i * BLK + lax.broadcasted_iota(I32, (BLK, 1), 0)
    ov = jnp.where(rows < N_F, o, 0.0)
    s1 = jnp.sum(ov, axis=0, keepdims=True)
    s2 = jnp.sum(ov * ov, axis=0, keepdims=True)

    @pl.when(i == 0)
    def _():
        st_ref[...] = jnp.zeros((8, C_OUT), F32)

    st_ref[0:1, :] += s1
    st_ref[1:2, :] += s2


def _f1(pself_f, agg_f, b2, bs):
    return pl.pallas_call(
        _f1_body,
        grid=(NBLK,),
        in_specs=[
            pl.BlockSpec((BLK, 2 * C_OUT), lambda i: (i, 0)),
            pl.BlockSpec((4, BLK, C_MID), lambda i: (0, i, 0)),
            pl.BlockSpec((1, C_OUT), lambda i: (0, 0)),
            pl.BlockSpec((1, C_OUT), lambda i: (0, 0)),
        ],
        out_specs=[
            pl.BlockSpec((BLK, C_OUT), lambda i: (i, 0)),
            pl.BlockSpec((8, C_OUT), lambda i: (0, 0)),
        ],
        out_shape=[
            jax.ShapeDtypeStruct((NFP, C_OUT), F32),
            jax.ShapeDtypeStruct((8, C_OUT), F32),
        ],
    )(pself_f, agg_f, b2, bs)


def _f3_body(o_ref, st_ref, g_ref, b_ref, out_ref):
    mean = st_ref[0:1, :] / N_F
    ex2 = st_ref[1:2, :] / N_F
    var = ex2 - mean * mean
    inv = lax.rsqrt(var + 1e-5)
    out_ref[...] = _selu((o_ref[...] - mean) * inv * g_ref[...] + b_ref[...])


def _f3(o, stats, gamma, beta):
    return pl.pallas_call(
        _f3_body,
        grid=(NBLK,),
        in_specs=[
            pl.BlockSpec((BLK, C_OUT), lambda i: (i, 0)),
            pl.BlockSpec((8, C_OUT), lambda i: (0, 0)),
            pl.BlockSpec((1, C_OUT), lambda i: (0, 0)),
            pl.BlockSpec((1, C_OUT), lambda i: (0, 0)),
        ],
        out_specs=pl.BlockSpec((BLK, C_OUT), lambda i: (i, 0)),
        out_shape=jax.ShapeDtypeStruct((NFP, C_OUT), F32),
    )(o, stats, gamma, beta)


CH = 128

_CC_PER_TILE = 39
_CC_BASE = 32 * _CC_PER_TILE * CH
_CF_PER_TILE = 156
_CF_BASE = 16 * _CF_PER_TILE * CH


def _edge_pipeline(n_chunks, src_off, dst_off, table, src_hbm, dst_hbm,
                   accum, bufs):
    def fire_idx(j, b):
        sv, dv, _, sa, _ = bufs[b]
        pltpu.async_copy(src_hbm.at[pl.ds(src_off(j), CH)], sv, sa)
        pltpu.async_copy(dst_hbm.at[pl.ds(dst_off(j), CH)], dv, sa)

    def wait_idx(b):
        sv, dv, _, sa, _ = bufs[b]
        pltpu.make_async_copy(src_hbm.at[pl.ds(0, CH)], sv, sa).wait()
        pltpu.make_async_copy(dst_hbm.at[pl.ds(0, CH)], dv, sa).wait()

    def fire_gather(b):
        sv, _, rv, _, sg = bufs[b]
        pltpu.async_copy(table.at[sv], rv, sg)

    def wait_gather(b):
        sv, _, rv, _, sg = bufs[b]
        pltpu.make_async_copy(table.at[sv], rv, sg).wait()

    def scatter(b):
        _, dv, rv, _, _ = bufs[b]
        pltpu.sync_copy(rv, accum.at[dv], add=True)

    n_even = n_chunks - (n_chunks % 2)
    fire_idx(0, 0)
    fire_idx(1, 1)
    wait_idx(0)
    fire_gather(0)
    last_t = n_even // 2 - 1

    def step(t, carry):
        wait_idx(1)
        fire_gather(1)
        wait_gather(0)
        scatter(0)

        @pl.when(t < last_t)
        def _():
            fire_idx(2 * t + 2, 0)
            wait_idx(0)
            fire_gather(0)

        wait_gather(1)
        scatter(1)

        @pl.when(t < last_t)
        def _():
            fire_idx(2 * t + 3, 1)

        return carry

    lax.fori_loop(0, n_even // 2, step, 0)
    if n_chunks % 2:
        fire_idx(n_chunks - 1, 0)
        wait_idx(0)
        fire_gather(0)
        wait_gather(0)
        scatter(0)


def _segsum_coarse_body(amsg, srcc, dstc, z, out, accum,
                        src_v0, dst_v0, rows_v0, src_v1, dst_v1, rows_v1,
                        semA0, semG0, semA1, semG1):
    c = lax.axis_index("c")
    s = lax.axis_index("s")
    w = c * 16 + s
    pltpu.sync_copy(z.at[pl.ds(0, 640)], accum.at[pl.ds(s * 640, 640)])
    plsc.subcore_barrier()
    base = w * (_CC_PER_TILE * CH)
    bufs = ((src_v0, dst_v0, rows_v0, semA0, semG0),
            (src_v1, dst_v1, rows_v1, semA1, semG1))
    _edge_pipeline(_CC_PER_TILE, lambda j: base + j * CH,
                   lambda j: base + j * CH, amsg, srcc, dstc, accum, bufs)

    @pl.when(s == 0)
    def _():
        off = _CC_BASE + c * CH
        pltpu.sync_copy(srcc.at[pl.ds(off, CH)], src_v0)
        pltpu.async_copy(amsg.at[src_v0], rows_v0, semG0).wait()
        pltpu.sync_copy(dstc.at[pl.ds(off, CH)], dst_v0)
        pltpu.sync_copy(rows_v0, accum.at[dst_v0], add=True)

    plsc.subcore_barrier()
    pltpu.sync_copy(accum.at[pl.ds(s * 640, 640)],
                    out.at[c, pl.ds(s * 640, 640)])


ICH = 64
_IQ = 1920 // ICH


def _interp_gather_body(ptab, idx3, gout,
                        idx_v0, rows_v0, idx_v1, rows_v1,
                        semA0, semG0, semA1, semG1):
    c = lax.axis_index("c")
    s = lax.axis_index("s")
    w = c * 16 + s
    bufs = ((idx_v0, rows_v0, semA0, semG0), (idx_v1, rows_v1, semA1, semG1))

    def korow(q):
        k = q // 10
        return k, w * 640 + (q - k * 10) * ICH

    def fire_idx(q, b):
        iv, _, sa, _ = bufs[b]
        k, row = korow(q)
        pltpu.async_copy(idx3.at[pl.ds(k * NFP + row, ICH)], iv, sa)

    def wait_idx(b):
        iv, _, sa, _ = bufs[b]
        pltpu.make_async_copy(idx3.at[pl.ds(0, ICH)], iv, sa).wait()

    def fire_gather(b):
        iv, rv, _, sg = bufs[b]
        pltpu.async_copy(ptab.at[iv], rv, sg)

    def wait_gather(b):
        iv, rv, _, sg = bufs[b]
        pltpu.make_async_copy(ptab.at[iv], rv, sg).wait()

    def writeback(q, b):
        _, rv, _, _ = bufs[b]
        k, row = korow(q)
        pltpu.sync_copy(rv, gout.at[k, pl.ds(row, ICH)])

    fire_idx(0, 0)
    fire_idx(1, 1)
    wait_idx(0)
    fire_gather(0)
    last_t = _IQ // 2 - 1

    def step(t, carry):
        wait_idx(1)
        fire_gather(1)
        wait_gather(0)
        writeback(2 * t, 0)

        @pl.when(t < last_t)
        def _():
            fire_idx(2 * t + 2, 0)
            wait_idx(0)
            fire_gather(0)

        wait_gather(1)
        writeback(2 * t + 1, 1)

        @pl.when(t < last_t)
        def _():
            fire_idx(2 * t + 3, 1)

        return carry

    lax.fori_loop(0, _IQ // 2, step, 0)


def _segsum_fine_body(pm, src4, dstf, z, out, accum,
                      src_v0, dst_v0, rows_v0, src_v1, dst_v1, rows_v1,
                      semA0, semG0, semA1, semG1):
    c = lax.axis_index("c")
    s = lax.axis_index("s")
    bufs = ((src_v0, dst_v0, rows_v0, semA0, semG0),
            (src_v1, dst_v1, rows_v1, semA1, semG1))
    base = s * (_CF_PER_TILE * CH)
    for gi in range(2):
        g = c * 2 + gi
        goff = g * E_F
        pltpu.sync_copy(z, accum.at[pl.ds(s * 1280, 1280)])
        plsc.subcore_barrier()
        _edge_pipeline(_CF_PER_TILE,
                       lambda j, goff=goff: goff + base + j * CH,
                       lambda j: base + j * CH, pm, src4, dstf, accum, bufs)

        @pl.when(s < 4)
        def _(goff=goff):
            off = _CF_BASE + s * CH
            pltpu.sync_copy(src4.at[pl.ds(goff + off, CH)], src_v0)
            pltpu.async_copy(pm.at[src_v0], rows_v0, semG0).wait()
            pltpu.sync_copy(dstf.at[pl.ds(off, CH)], dst_v0)
            pltpu.sync_copy(rows_v0, accum.at[dst_v0], add=True)

        plsc.subcore_barrier()
        pltpu.sync_copy(accum.at[pl.ds(s * 1280, 1280)],
                        out.at[g, pl.ds(s * 1280, 1280)])
        plsc.subcore_barrier()


@functools.lru_cache(maxsize=1)
def _sc_kernels():
    mesh = plsc.VectorSubcoreMesh(core_axis_name="c", subcore_axis_name="s")
    params = pltpu.CompilerParams(use_tc_tiling_on_sc=False)
    edge_scratch = [
        pltpu.VMEM((CH,), I32),
        pltpu.VMEM((CH,), I32),
        pltpu.VMEM((CH, C_MID), F32),
        pltpu.VMEM((CH,), I32),
        pltpu.VMEM((CH,), I32),
        pltpu.VMEM((CH, C_MID), F32),
        pltpu.SemaphoreType.DMA,
        pltpu.SemaphoreType.DMA,
        pltpu.SemaphoreType.DMA,
        pltpu.SemaphoreType.DMA,
    ]
    segsum_coarse = pl.kernel(
        _segsum_coarse_body,
        out_type=jax.ShapeDtypeStruct((2, NCP, C_MID), F32),
        mesh=mesh,
        scratch_types=[pltpu.VMEM_SHARED((NCP, C_MID), F32)] + edge_scratch,
        compiler_params=params,
    )
    interp_gather = pl.kernel(
        _interp_gather_body,
        out_type=jax.ShapeDtypeStruct((3, NFP, 4 * C_OUT), F32),
        mesh=mesh,
        scratch_types=[
            pltpu.VMEM((ICH,), I32),
            pltpu.VMEM((ICH, 4 * C_OUT), F32),
            pltpu.VMEM((ICH,), I32),
            pltpu.VMEM((ICH, 4 * C_OUT), F32),
            pltpu.SemaphoreType.DMA,
            pltpu.SemaphoreType.DMA,
            pltpu.SemaphoreType.DMA,
            pltpu.SemaphoreType.DMA,
        ],
        compiler_params=params,
    )
    segsum_fine = pl.kernel(
        _segsum_fine_body,
        out_type=jax.ShapeDtypeStruct((4, NFP, C_MID), F32),
        mesh=mesh,
        scratch_types=[pltpu.VMEM_SHARED((NFP, C_MID), F32)] + edge_scratch,
        compiler_params=params,
    )
    return segsum_coarse, interp_gather, segsum_fine


def kernel(x, mesh_pos, m_pos_new, W_self1, W_msg1, b1, W_self2, W_msg2, b2,
           Ws_self, Ws_msg, bs, gamma, beta, edge_index, edge_index_fine):
    psn = jnp.pad(mesh_pos.astype(F32), ((0, NCP - N_C), (0, 5)),
                  constant_values=1e6)
    pd_blocks = (jnp.pad(m_pos_new.astype(F32), ((0, NFP - N_F), (0, 5)))
                 .reshape(KNBLK, KBLK, 8).transpose(0, 2, 1))
    src_c = edge_index[0].astype(I32)
    dst_c = edge_index[1].astype(I32)
    src_f = edge_index_fine[0].astype(I32)
    dst_f = edge_index_fine[1].astype(I32)
    src4 = (src_f[None, :]
            + (jnp.arange(4, dtype=I32) * NFP)[:, None]).reshape(4 * E_F)
    z = jnp.zeros((1280, C_MID), F32)
    _segsum_coarse, _interp_gather, _segsum_fine = _sc_kernels()

    idxo, wno = _knn(psn, pd_blocks)
    idx3 = idxo.transpose(1, 0, 2).reshape(3 * NFP)
    wn8 = jnp.pad(wno.transpose(0, 2, 1).reshape(NFP, 3), ((0, 0), (0, 5)))

    a_self, a_msg = _m1(x, W_self1, W_msg1)
    agg_c = _segsum_coarse(a_msg, src_c, dst_c, z)
    p = _p_kernel(a_self, agg_c, b1.reshape(1, C_MID), x,
                  W_self2, W_msg2, Ws_self, Ws_msg)

    gtab = _interp_gather(p, idx3)
    pself_f, pmsg = _w_kernel(gtab, wn8)

    agg_f = _segsum_fine(pmsg.reshape(4 * NFP, C_MID), src4, dst_f, z)

    o, stats = _f1(pself_f, agg_f, b2.reshape(1, C_OUT), bs.reshape(1, C_OUT))
    out = _f3(o, stats, gamma.reshape(1, C_OUT), beta.reshape(1, C_OUT))
    return out[:N_F]


# speedup vs baseline: 1.0286x; 1.0286x over previous
import functools

import jax
import jax.numpy as jnp
from jax import lax
from jax.experimental import pallas as pl
from jax.experimental.pallas import tpu as pltpu
from jax.experimental.pallas import tpu_sc as plsc

F32 = jnp.float32
I32 = jnp.int32

N_C = 10000
N_F = 20000
NFP = 20480
NCP = 10240
C_IN = 128
C_MID = 64
C_OUT = 128
E_C = 160000
E_F = 320000

BLK = 512
NBLK = NFP // BLK

SELU_ALPHA = 1.6732632423543772
SELU_SCALE = 1.0507009873554805


def _selu(v):
    return SELU_SCALE * jnp.where(v > 0, v, SELU_ALPHA * (jnp.exp(v) - 1.0))


KBLK = 512
KNBLK = NFP // KBLK
_BIG_I = NCP
_INF = 3e38


def _knn_body(ps_ref, pd_ref, idx_ref, wn_ref, d2_ref):
    acc = None
    for d in range(3):
        ps_d = ps_ref[:, d:d + 1]
        pd_d = pd_ref[0, d:d + 1, :]
        t = ps_d - pd_d
        t = t * t
        acc = t if acc is None else acc + t
    d2_ref[...] = acc
    iota = lax.broadcasted_iota(I32, (NCP, KBLK), 0)
    t = d2_ref[...]
    m1 = jnp.min(t, axis=0, keepdims=True)
    a1 = jnp.min(jnp.where(t == m1, iota, _BIG_I), axis=0, keepdims=True)
    not1 = iota != a1
    m2 = jnp.min(jnp.where(not1, t, _INF), axis=0, keepdims=True)
    a2 = jnp.min(jnp.where((t == m2) & not1, iota, _BIG_I), axis=0,
                 keepdims=True)
    not12 = not1 & (iota != a2)
    m3 = jnp.min(jnp.where(not12, t, _INF), axis=0, keepdims=True)
    a3 = jnp.min(jnp.where((t == m3) & not12, iota, _BIG_I), axis=0,
                 keepdims=True)
    w1 = 1.0 / (jnp.maximum(m1, 0.0) + 1e-8)
    w2 = 1.0 / (jnp.maximum(m2, 0.0) + 1e-8)
    w3 = 1.0 / (jnp.maximum(m3, 0.0) + 1e-8)
    s = w1 + w2 + w3
    for k, (a, w) in enumerate(((a1, w1), (a2, w2), (a3, w3))):
        idx_ref[0, k:k + 1, :] = a
        wn_ref[0, k:k + 1, :] = w / s


def _knn(ps_pad, pd_blocks):
    return pl.pallas_call(
        _knn_body,
        grid=(KNBLK,),
        in_specs=[
            pl.BlockSpec((NCP, 8), lambda i: (0, 0)),
            pl.BlockSpec((1, 8, KBLK), lambda i: (i, 0, 0)),
        ],
        out_specs=[
            pl.BlockSpec((1, 3, KBLK), lambda i: (i, 0, 0)),
            pl.BlockSpec((1, 3, KBLK), lambda i: (i, 0, 0)),
        ],
        out_shape=[
            jax.ShapeDtypeStruct((KNBLK, 3, KBLK), I32),
            jax.ShapeDtypeStruct((KNBLK, 3, KBLK), F32),
        ],
        scratch_shapes=[pltpu.VMEM((NCP, KBLK), F32)],
    )(ps_pad, pd_blocks)


M1_BLK = 1000


def _m1_body(x_ref, ws_ref, wm_ref, as_ref, am_ref):
    xb = x_ref[...]
    as_ref[...] = jnp.dot(xb, ws_ref[...], preferred_element_type=F32)
    am_ref[...] = jnp.dot(xb, wm_ref[...], preferred_element_type=F32)


def _m1(x, ws, wm):
    return pl.pallas_call(
        _m1_body,
        grid=(N_C // M1_BLK,),
        in_specs=[
            pl.BlockSpec((M1_BLK, C_IN), lambda i: (i, 0)),
            pl.BlockSpec((C_IN, C_MID), lambda i: (0, 0)),
            pl.BlockSpec((C_IN, C_MID), lambda i: (0, 0)),
        ],
        out_specs=[
            pl.BlockSpec((M1_BLK, C_MID), lambda i: (i, 0)),
            pl.BlockSpec((M1_BLK, C_MID), lambda i: (i, 0)),
        ],
        out_shape=[jax.ShapeDtypeStruct((N_C, C_MID), F32)] * 2,
    )(x, ws, wm)


def _p_body(as_ref, agg_ref, b1_ref, x_ref, w2s_ref, w2m_ref, wss_ref,
            wsm_ref, p_ref):
    h = _selu(as_ref[...] + agg_ref[0] + agg_ref[1] + b1_ref[...])
    xb = x_ref[...]
    p_ref[...] = jnp.concatenate(
        [
            jnp.dot(h, w2s_ref[...], preferred_element_type=F32),
            jnp.dot(xb, wss_ref[...], preferred_element_type=F32),
            jnp.dot(h, w2m_ref[...], preferred_element_type=F32),
            jnp.dot(xb, wsm_ref[...], preferred_element_type=F32),
        ],
        axis=1,
    )


def _p_kernel(a_self, agg_c, b1, x, w2s, w2m, wss, wsm):
    return pl.pallas_call(
        _p_body,
        grid=(N_C // M1_BLK,),
        in_specs=[
            pl.BlockSpec((M1_BLK, C_MID), lambda i: (i, 0)),
            pl.BlockSpec((2, M1_BLK, C_MID), lambda i: (0, i, 0)),
            pl.BlockSpec((1, C_MID), lambda i: (0, 0)),
            pl.BlockSpec((M1_BLK, C_IN), lambda i: (i, 0)),
            pl.BlockSpec((C_MID, C_OUT), lambda i: (0, 0)),
            pl.BlockSpec((C_MID, C_OUT), lambda i: (0, 0)),
            pl.BlockSpec((C_IN, C_OUT), lambda i: (0, 0)),
            pl.BlockSpec((C_IN, C_OUT), lambda i: (0, 0)),
        ],
        out_specs=pl.BlockSpec((M1_BLK, 4 * C_OUT), lambda i: (i, 0)),
        out_shape=jax.ShapeDtypeStruct((N_C, 4 * C_OUT), F32),
    )(a_self, agg_c, b1, x, w2s, w2m, wss, wsm)


def _w_body(g_ref, wn_ref, pself_ref, pmsg_ref):
    p = None
    for k in range(3):
        wk = wn_ref[:, k:k + 1]
        t = g_ref[k] * wk
        p = t if p is None else p + t
    pself_ref[...] = p[:, :2 * C_OUT]
    for g in range(4):
        pmsg_ref[g] = p[:, 2 * C_OUT + C_MID * g: 2 * C_OUT + C_MID * (g + 1)]


def _w_kernel(gtab, wn8):
    return pl.pallas_call(
        _w_body,
        grid=(NBLK,),
        in_specs=[
            pl.BlockSpec((3, BLK, 4 * C_OUT), lambda i: (0, i, 0)),
            pl.BlockSpec((BLK, 8), lambda i: (i, 0)),
        ],
        out_specs=[
            pl.BlockSpec((BLK, 2 * C_OUT), lambda i: (i, 0)),
            pl.BlockSpec((4, BLK, C_MID), lambda i: (0, i, 0)),
        ],
        out_shape=[
            jax.ShapeDtypeStruct((NFP, 2 * C_OUT), F32),
            jax.ShapeDtypeStruct((4, NFP, C_MID), F32),
        ],
    )(gtab, wn8)


def _f1_body(ps_ref, ag_ref, b2_ref, bs_ref, o_ref, st_ref):
    i = pl.program_id(0)
    ms = ps_ref[:, :C_OUT]
    ss = ps_ref[:, C_OUT:]
    am = jnp.concatenate([ag_ref[0], ag_ref[1]], axis=1)
    ak = jnp.concatenate([ag_ref[2], ag_ref[3]], axis=1)
    o = _selu(ms + am + b2_ref[...]) + _selu(ss + ak + bs_ref[...])
    o_ref[...] = o
    rows = i * BLK + lax.broadcasted_iota(I32, (BLK, 1), 0)
    ov = jnp.where(rows < N_F, o, 0.0)
    s1 = jnp.sum(ov, axis=0, keepdims=True)
    s2 = jnp.sum(ov * ov, axis=0, keepdims=True)

    @pl.when(i == 0)
    def _():
        st_ref[...] = jnp.zeros((8, C_OUT), F32)

    st_ref[0:1, :] += s1
    st_ref[1:2, :] += s2


def _f1(pself_f, agg_f, b2, bs):
    return pl.pallas_call(
        _f1_body,
        grid=(NBLK,),
        in_specs=[
            pl.BlockSpec((BLK, 2 * C_OUT), lambda i: (i, 0)),
            pl.BlockSpec((4, BLK, C_MID), lambda i: (0, i, 0)),
            pl.BlockSpec((1, C_OUT), lambda i: (0, 0)),
            pl.BlockSpec((1, C_OUT), lambda i: (0, 0)),
        ],
        out_specs=[
            pl.BlockSpec((BLK, C_OUT), lambda i: (i, 0)),
            pl.BlockSpec((8, C_OUT), lambda i: (0, 0)),
        ],
        out_shape=[
            jax.ShapeDtypeStruct((NFP, C_OUT), F32),
            jax.ShapeDtypeStruct((8, C_OUT), F32),
        ],
    )(pself_f, agg_f, b2, bs)


def _f3_body(o_ref, st_ref, g_ref, b_ref, out_ref):
    mean = st_ref[0:1, :] / N_F
    ex2 = st_ref[1:2, :] / N_F
    var = ex2 - mean * mean
    inv = lax.rsqrt(var + 1e-5)
    out_ref[...] = _selu((o_ref[...] - mean) * inv * g_ref[...] + b_ref[...])


def _f3(o, stats, gamma, beta):
    return pl.pallas_call(
        _f3_body,
        grid=(NBLK,),
        in_specs=[
            pl.BlockSpec((BLK, C_OUT), lambda i: (i, 0)),
            pl.BlockSpec((8, C_OUT), lambda i: (0, 0)),
            pl.BlockSpec((1, C_OUT), lambda i: (0, 0)),
            pl.BlockSpec((1, C_OUT), lambda i: (0, 0)),
        ],
        out_specs=pl.BlockSpec((BLK, C_OUT), lambda i: (i, 0)),
        out_shape=jax.ShapeDtypeStruct((NFP, C_OUT), F32),
    )(o, stats, gamma, beta)


CH = 128

_CC_PER_TILE = 39
_CC_BASE = 32 * _CC_PER_TILE * CH
_CF_PER_TILE = 156
_CF_BASE = 16 * _CF_PER_TILE * CH


def _edge_pipeline(n_chunks, src_off, dst_off, table, src_hbm, dst_hbm,
                   accum, bufs):
    def fire_idx(j, b):
        sv, dv, _, sa, _ = bufs[b]
        pltpu.async_copy(src_hbm.at[pl.ds(src_off(j), CH)], sv, sa)
        pltpu.async_copy(dst_hbm.at[pl.ds(dst_off(j), CH)], dv, sa)

    def wait_idx(b):
        sv, dv, _, sa, _ = bufs[b]
        pltpu.make_async_copy(src_hbm.at[pl.ds(0, CH)], sv, sa).wait()
        pltpu.make_async_copy(dst_hbm.at[pl.ds(0, CH)], dv, sa).wait()

    def fire_gather(b):
        sv, _, rv, _, sg = bufs[b]
        pltpu.async_copy(table.at[sv], rv, sg)

    def wait_gather(b):
        sv, _, rv, _, sg = bufs[b]
        pltpu.make_async_copy(table.at[sv], rv, sg).wait()

    def scatter(b):
        _, dv, rv, _, _ = bufs[b]
        pltpu.sync_copy(rv, accum.at[dv], add=True)

    n_even = n_chunks - (n_chunks % 2)
    fire_idx(0, 0)
    fire_idx(1, 1)
    wait_idx(0)
    fire_gather(0)
    last_t = n_even // 2 - 1

    def step(t, carry):
        wait_idx(1)
        fire_gather(1)
        wait_gather(0)
        scatter(0)

        @pl.when(t < last_t)
        def _():
            fire_idx(2 * t + 2, 0)
            wait_idx(0)
            fire_gather(0)

        wait_gather(1)
        scatter(1)

        @pl.when(t < last_t)
        def _():
            fire_idx(2 * t + 3, 1)

        return carry

    lax.fori_loop(0, n_even // 2, step, 0)
    if n_chunks % 2:
        fire_idx(n_chunks - 1, 0)
        wait_idx(0)
        fire_gather(0)
        wait_gather(0)
        scatter(0)


def _segsum_coarse_body(amsg, srcc, dstc, z, out, accum,
                        src_v0, dst_v0, rows_v0, src_v1, dst_v1, rows_v1,
                        semA0, semG0, semA1, semG1):
    c = lax.axis_index("c")
    s = lax.axis_index("s")
    w = c * 16 + s
    pltpu.sync_copy(z.at[pl.ds(0, 640)], accum.at[pl.ds(s * 640, 640)])
    plsc.subcore_barrier()
    base = w * (_CC_PER_TILE * CH)
    bufs = ((src_v0, dst_v0, rows_v0, semA0, semG0),
            (src_v1, dst_v1, rows_v1, semA1, semG1))
    _edge_pipeline(_CC_PER_TILE, lambda j: base + j * CH,
                   lambda j: base + j * CH, amsg, srcc, dstc, accum, bufs)

    @pl.when(s == 0)
    def _():
        off = _CC_BASE + c * CH
        pltpu.sync_copy(srcc.at[pl.ds(off, CH)], src_v0)
        pltpu.async_copy(amsg.at[src_v0], rows_v0, semG0).wait()
        pltpu.sync_copy(dstc.at[pl.ds(off, CH)], dst_v0)
        pltpu.sync_copy(rows_v0, accum.at[dst_v0], add=True)

    plsc.subcore_barrier()
    pltpu.sync_copy(accum.at[pl.ds(s * 640, 640)],
                    out.at[c, pl.ds(s * 640, 640)])


ICH = 80
_IQ = 1920 // ICH


def _interp_gather_body(ptab, idx3, gout,
                        idx_v0, rows_v0, idx_v1, rows_v1,
                        semA0, semG0, semA1, semG1):
    c = lax.axis_index("c")
    s = lax.axis_index("s")
    w = c * 16 + s
    bufs = ((idx_v0, rows_v0, semA0, semG0), (idx_v1, rows_v1, semA1, semG1))

    ipk = 640 // ICH

    def korow(q):
        k = q // ipk
        return k, w * 640 + (q - k * ipk) * ICH

    def fire_idx(q, b):
        iv, _, sa, _ = bufs[b]
        k, row = korow(q)
        pltpu.async_copy(idx3.at[pl.ds(k * NFP + row, ICH)], iv, sa)

    def wait_idx(b):
        iv, _, sa, _ = bufs[b]
        pltpu.make_async_copy(idx3.at[pl.ds(0, ICH)], iv, sa).wait()

    def fire_gather(b):
        iv, rv, _, sg = bufs[b]
        pltpu.async_copy(ptab.at[iv], rv, sg)

    def wait_gather(b):
        iv, rv, _, sg = bufs[b]
        pltpu.make_async_copy(ptab.at[iv], rv, sg).wait()

    def writeback(q, b):
        _, rv, _, _ = bufs[b]
        k, row = korow(q)
        pltpu.sync_copy(rv, gout.at[k, pl.ds(row, ICH)])

    fire_idx(0, 0)
    fire_idx(1, 1)
    wait_idx(0)
    fire_gather(0)
    last_t = _IQ // 2 - 1

    def step(t, carry):
        wait_idx(1)
        fire_gather(1)
        wait_gather(0)
        writeback(2 * t, 0)

        @pl.when(t < last_t)
        def _():
            fire_idx(2 * t + 2, 0)
            wait_idx(0)
            fire_gather(0)

        wait_gather(1)
        writeback(2 * t + 1, 1)

        @pl.when(t < last_t)
        def _():
            fire_idx(2 * t + 3, 1)

        return carry

    lax.fori_loop(0, _IQ // 2, step, 0)


def _segsum_fine_body(pm, src4, dstf, z, out, accum,
                      src_v0, dst_v0, rows_v0, src_v1, dst_v1, rows_v1,
                      semA0, semG0, semA1, semG1):
    c = lax.axis_index("c")
    s = lax.axis_index("s")
    bufs = ((src_v0, dst_v0, rows_v0, semA0, semG0),
            (src_v1, dst_v1, rows_v1, semA1, semG1))
    base = s * (_CF_PER_TILE * CH)
    for gi in range(2):
        g = c * 2 + gi
        goff = g * E_F
        pltpu.sync_copy(z, accum.at[pl.ds(s * 1280, 1280)])
        plsc.subcore_barrier()
        _edge_pipeline(_CF_PER_TILE,
                       lambda j, goff=goff: goff + base + j * CH,
                       lambda j: base + j * CH, pm, src4, dstf, accum, bufs)

        @pl.when(s < 4)
        def _(goff=goff):
            off = _CF_BASE + s * CH
            pltpu.sync_copy(src4.at[pl.ds(goff + off, CH)], src_v0)
            pltpu.async_copy(pm.at[src_v0], rows_v0, semG0).wait()
            pltpu.sync_copy(dstf.at[pl.ds(off, CH)], dst_v0)
            pltpu.sync_copy(rows_v0, accum.at[dst_v0], add=True)

        plsc.subcore_barrier()
        pltpu.sync_copy(accum.at[pl.ds(s * 1280, 1280)],
                        out.at[g, pl.ds(s * 1280, 1280)])
        plsc.subcore_barrier()


@functools.lru_cache(maxsize=1)
def _sc_kernels():
    mesh = plsc.VectorSubcoreMesh(core_axis_name="c", subcore_axis_name="s")
    params = pltpu.CompilerParams(use_tc_tiling_on_sc=False)
    edge_scratch = [
        pltpu.VMEM((CH,), I32),
        pltpu.VMEM((CH,), I32),
        pltpu.VMEM((CH, C_MID), F32),
        pltpu.VMEM((CH,), I32),
        pltpu.VMEM((CH,), I32),
        pltpu.VMEM((CH, C_MID), F32),
        pltpu.SemaphoreType.DMA,
        pltpu.SemaphoreType.DMA,
        pltpu.SemaphoreType.DMA,
        pltpu.SemaphoreType.DMA,
    ]
    segsum_coarse = pl.kernel(
        _segsum_coarse_body,
        out_type=jax.ShapeDtypeStruct((2, NCP, C_MID), F32),
        mesh=mesh,
        scratch_types=[pltpu.VMEM_SHARED((NCP, C_MID), F32)] + edge_scratch,
        compiler_params=params,
    )
    interp_gather = pl.kernel(
        _interp_gather_body,
        out_type=jax.ShapeDtypeStruct((3, NFP, 4 * C_OUT), F32),
        mesh=mesh,
        scratch_types=[
            pltpu.VMEM((ICH,), I32),
            pltpu.VMEM((ICH, 4 * C_OUT), F32),
            pltpu.VMEM((ICH,), I32),
            pltpu.VMEM((ICH, 4 * C_OUT), F32),
            pltpu.SemaphoreType.DMA,
            pltpu.SemaphoreType.DMA,
            pltpu.SemaphoreType.DMA,
            pltpu.SemaphoreType.DMA,
        ],
        compiler_params=params,
    )
    segsum_fine = pl.kernel(
        _segsum_fine_body,
        out_type=jax.ShapeDtypeStruct((4, NFP, C_MID), F32),
        mesh=mesh,
        scratch_types=[pltpu.VMEM_SHARED((NFP, C_MID), F32)] + edge_scratch,
        compiler_params=params,
    )
    return segsum_coarse, interp_gather, segsum_fine


def kernel(x, mesh_pos, m_pos_new, W_self1, W_msg1, b1, W_self2, W_msg2, b2,
           Ws_self, Ws_msg, bs, gamma, beta, edge_index, edge_index_fine):
    psn = jnp.pad(mesh_pos.astype(F32), ((0, NCP - N_C), (0, 5)),
                  constant_values=1e6)
    pd_blocks = (jnp.pad(m_pos_new.astype(F32), ((0, NFP - N_F), (0, 5)))
                 .reshape(KNBLK, KBLK, 8).transpose(0, 2, 1))
    src_c = edge_index[0].astype(I32)
    dst_c = edge_index[1].astype(I32)
    src_f = edge_index_fine[0].astype(I32)
    dst_f = edge_index_fine[1].astype(I32)
    src4 = (src_f[None, :]
            + (jnp.arange(4, dtype=I32) * NFP)[:, None]).reshape(4 * E_F)
    z = jnp.zeros((1280, C_MID), F32)
    _segsum_coarse, _interp_gather, _segsum_fine = _sc_kernels()

    idxo, wno = _knn(psn, pd_blocks)
    idx3 = idxo.transpose(1, 0, 2).reshape(3 * NFP)
    wn8 = jnp.pad(wno.transpose(0, 2, 1).reshape(NFP, 3), ((0, 0), (0, 5)))

    a_self, a_msg = _m1(x, W_self1, W_msg1)
    agg_c = _segsum_coarse(a_msg, src_c, dst_c, z)
    p = _p_kernel(a_self, agg_c, b1.reshape(1, C_MID), x,
                  W_self2, W_msg2, Ws_self, Ws_msg)

    gtab = _interp_gather(p, idx3)
    pself_f, pmsg = _w_kernel(gtab, wn8)

    agg_f = _segsum_fine(pmsg.reshape(4 * NFP, C_MID), src4, dst_f, z)

    o, stats = _f1(pself_f, agg_f, b2.reshape(1, C_OUT), bs.reshape(1, C_OUT))
    out = _f3(o, stats, gamma.reshape(1, C_OUT), beta.reshape(1, C_OUT))
    return out[:N_F]


# speedup vs baseline: 1.0610x; 1.0316x over previous
import functools

import jax
import jax.numpy as jnp
from jax import lax
from jax.experimental import pallas as pl
from jax.experimental.pallas import tpu as pltpu
from jax.experimental.pallas import tpu_sc as plsc

F32 = jnp.float32
I32 = jnp.int32

N_C = 10000
N_F = 20000
NFP = 20480
NCP = 10240
C_IN = 128
C_MID = 64
C_OUT = 128
E_C = 160000
E_F = 320000

BLK = 512
NBLK = NFP // BLK

SELU_ALPHA = 1.6732632423543772
SELU_SCALE = 1.0507009873554805


def _selu(v):
    return SELU_SCALE * jnp.where(v > 0, v, SELU_ALPHA * (jnp.exp(v) - 1.0))


KBLK = 640
KNBLK = NFP // KBLK
NKP = 10112
_BIG_I = NCP
_INF = 3e38


def _knn_body(ps_ref, pd_ref, idx_ref, wn_ref, d2_ref):
    acc = None
    for d in range(3):
        ps_d = ps_ref[:, d:d + 1]
        pd_d = pd_ref[0, d:d + 1, :]
        t = ps_d - pd_d
        t = t * t
        acc = t if acc is None else acc + t
    d2_ref[...] = acc
    iota = lax.broadcasted_iota(I32, (NKP, KBLK), 0)
    t = d2_ref[...]
    m1 = jnp.min(t, axis=0, keepdims=True)
    a1 = jnp.min(jnp.where(t == m1, iota, _BIG_I), axis=0, keepdims=True)
    not1 = iota != a1
    m2 = jnp.min(jnp.where(not1, t, _INF), axis=0, keepdims=True)
    a2 = jnp.min(jnp.where((t == m2) & not1, iota, _BIG_I), axis=0,
                 keepdims=True)
    not12 = not1 & (iota != a2)
    m3 = jnp.min(jnp.where(not12, t, _INF), axis=0, keepdims=True)
    a3 = jnp.min(jnp.where((t == m3) & not12, iota, _BIG_I), axis=0,
                 keepdims=True)
    w1 = 1.0 / (jnp.maximum(m1, 0.0) + 1e-8)
    w2 = 1.0 / (jnp.maximum(m2, 0.0) + 1e-8)
    w3 = 1.0 / (jnp.maximum(m3, 0.0) + 1e-8)
    s = w1 + w2 + w3
    for k, (a, w) in enumerate(((a1, w1), (a2, w2), (a3, w3))):
        idx_ref[0, k:k + 1, :] = a
        wn_ref[0, k:k + 1, :] = w / s


def _knn(ps_pad, pd_blocks):
    return pl.pallas_call(
        _knn_body,
        grid=(KNBLK,),
        in_specs=[
            pl.BlockSpec((NKP, 8), lambda i: (0, 0)),
            pl.BlockSpec((1, 8, KBLK), lambda i: (i, 0, 0)),
        ],
        out_specs=[
            pl.BlockSpec((1, 3, KBLK), lambda i: (i, 0, 0)),
            pl.BlockSpec((1, 3, KBLK), lambda i: (i, 0, 0)),
        ],
        out_shape=[
            jax.ShapeDtypeStruct((KNBLK, 3, KBLK), I32),
            jax.ShapeDtypeStruct((KNBLK, 3, KBLK), F32),
        ],
        scratch_shapes=[pltpu.VMEM((NKP, KBLK), F32)],
    )(ps_pad, pd_blocks)


M1_BLK = 1000


def _m1_body(x_ref, ws_ref, wm_ref, as_ref, am_ref):
    xb = x_ref[...]
    as_ref[...] = jnp.dot(xb, ws_ref[...], preferred_element_type=F32)
    am_ref[...] = jnp.dot(xb, wm_ref[...], preferred_element_type=F32)


def _m1(x, ws, wm):
    return pl.pallas_call(
        _m1_body,
        grid=(N_C // M1_BLK,),
        in_specs=[
            pl.BlockSpec((M1_BLK, C_IN), lambda i: (i, 0)),
            pl.BlockSpec((C_IN, C_MID), lambda i: (0, 0)),
            pl.BlockSpec((C_IN, C_MID), lambda i: (0, 0)),
        ],
        out_specs=[
            pl.BlockSpec((M1_BLK, C_MID), lambda i: (i, 0)),
            pl.BlockSpec((M1_BLK, C_MID), lambda i: (i, 0)),
        ],
        out_shape=[jax.ShapeDtypeStruct((N_C, C_MID), F32)] * 2,
    )(x, ws, wm)


def _p_body(as_ref, agg_ref, b1_ref, x_ref, w2s_ref, w2m_ref, wss_ref,
            wsm_ref, p_ref):
    h = _selu(as_ref[...] + agg_ref[0] + agg_ref[1] + b1_ref[...])
    xb = x_ref[...]
    p_ref[...] = jnp.concatenate(
        [
            jnp.dot(h, w2s_ref[...], preferred_element_type=F32),
            jnp.dot(xb, wss_ref[...], preferred_element_type=F32),
            jnp.dot(h, w2m_ref[...], preferred_element_type=F32),
            jnp.dot(xb, wsm_ref[...], preferred_element_type=F32),
        ],
        axis=1,
    )


def _p_kernel(a_self, agg_c, b1, x, w2s, w2m, wss, wsm):
    return pl.pallas_call(
        _p_body,
        grid=(N_C // M1_BLK,),
        in_specs=[
            pl.BlockSpec((M1_BLK, C_MID), lambda i: (i, 0)),
            pl.BlockSpec((2, M1_BLK, C_MID), lambda i: (0, i, 0)),
            pl.BlockSpec((1, C_MID), lambda i: (0, 0)),
            pl.BlockSpec((M1_BLK, C_IN), lambda i: (i, 0)),
            pl.BlockSpec((C_MID, C_OUT), lambda i: (0, 0)),
            pl.BlockSpec((C_MID, C_OUT), lambda i: (0, 0)),
            pl.BlockSpec((C_IN, C_OUT), lambda i: (0, 0)),
            pl.BlockSpec((C_IN, C_OUT), lambda i: (0, 0)),
        ],
        out_specs=pl.BlockSpec((M1_BLK, 4 * C_OUT), lambda i: (i, 0)),
        out_shape=jax.ShapeDtypeStruct((N_C, 4 * C_OUT), F32),
    )(a_self, agg_c, b1, x, w2s, w2m, wss, wsm)


def _w_body(g_ref, wn_ref, pself_ref, pmsg_ref):
    p = None
    for k in range(3):
        wk = wn_ref[:, k:k + 1]
        t = g_ref[k] * wk
        p = t if p is None else p + t
    pself_ref[...] = p[:, :2 * C_OUT]
    for g in range(4):
        pmsg_ref[g] = p[:, 2 * C_OUT + C_MID * g: 2 * C_OUT + C_MID * (g + 1)]


def _w_kernel(gtab, wn8):
    return pl.pallas_call(
        _w_body,
        grid=(NBLK,),
        in_specs=[
            pl.BlockSpec((3, BLK, 4 * C_OUT), lambda i: (0, i, 0)),
            pl.BlockSpec((BLK, 8), lambda i: (i, 0)),
        ],
        out_specs=[
            pl.BlockSpec((BLK, 2 * C_OUT), lambda i: (i, 0)),
            pl.BlockSpec((4, BLK, C_MID), lambda i: (0, i, 0)),
        ],
        out_shape=[
            jax.ShapeDtypeStruct((NFP, 2 * C_OUT), F32),
            jax.ShapeDtypeStruct((4, NFP, C_MID), F32),
        ],
    )(gtab, wn8)


def _f1_body(ps_ref, ag_ref, b2_ref, bs_ref, o_ref, st_ref):
    i = pl.program_id(0)
    ms = ps_ref[:, :C_OUT]
    ss = ps_ref[:, C_OUT:]
    am = jnp.concatenate([ag_ref[0], ag_ref[1]], axis=1)
    ak = jnp.concatenate([ag_ref[2], ag_ref[3]], axis=1)
    o = _selu(ms + am + b2_ref[...]) + _selu(ss + ak + bs_ref[...])
    o_ref[...] = o
    rows = i * BLK + lax.broadcasted_iota(I32, (BLK, 1), 0)
    ov = jnp.where(rows < N_F, o, 0.0)
    s1 = jnp.sum(ov, axis=0, keepdims=True)
    s2 = jnp.sum(ov * ov, axis=0, keepdims=True)

    @pl.when(i == 0)
    def _():
        st_ref[...] = jnp.zeros((8, C_OUT), F32)

    st_ref[0:1, :] += s1
    st_ref[1:2, :] += s2


def _f1(pself_f, agg_f, b2, bs):
    return pl.pallas_call(
        _f1_body,
        grid=(NBLK,),
        in_specs=[
            pl.BlockSpec((BLK, 2 * C_OUT), lambda i: (i, 0)),
            pl.BlockSpec((4, BLK, C_MID), lambda i: (0, i, 0)),
            pl.BlockSpec((1, C_OUT), lambda i: (0, 0)),
            pl.BlockSpec((1, C_OUT), lambda i: (0, 0)),
        ],
        out_specs=[
            pl.BlockSpec((BLK, C_OUT), lambda i: (i, 0)),
            pl.BlockSpec((8, C_OUT), lambda i: (0, 0)),
        ],
        out_shape=[
            jax.ShapeDtypeStruct((NFP, C_OUT), F32),
            jax.ShapeDtypeStruct((8, C_OUT), F32),
        ],
    )(pself_f, agg_f, b2, bs)


def _f3_body(o_ref, st_ref, g_ref, b_ref, out_ref):
    mean = st_ref[0:1, :] / N_F
    ex2 = st_ref[1:2, :] / N_F
    var = ex2 - mean * mean
    inv = lax.rsqrt(var + 1e-5)
    out_ref[...] = _selu((o_ref[...] - mean) * inv * g_ref[...] + b_ref[...])


def _f3(o, stats, gamma, beta):
    return pl.pallas_call(
        _f3_body,
        grid=(NBLK,),
        in_specs=[
            pl.BlockSpec((BLK, C_OUT), lambda i: (i, 0)),
            pl.BlockSpec((8, C_OUT), lambda i: (0, 0)),
            pl.BlockSpec((1, C_OUT), lambda i: (0, 0)),
            pl.BlockSpec((1, C_OUT), lambda i: (0, 0)),
        ],
        out_specs=pl.BlockSpec((BLK, C_OUT), lambda i: (i, 0)),
        out_shape=jax.ShapeDtypeStruct((NFP, C_OUT), F32),
    )(o, stats, gamma, beta)


CH = 128

_CC_PER_TILE = 39
_CC_BASE = 32 * _CC_PER_TILE * CH
_CF_PER_TILE = 156
_CF_BASE = 16 * _CF_PER_TILE * CH


def _edge_pipeline(n_chunks, src_off, dst_off, table, src_hbm, dst_hbm,
                   accum, bufs):
    def fire_idx(j, b):
        sv, dv, _, sa, _ = bufs[b]
        pltpu.async_copy(src_hbm.at[pl.ds(src_off(j), CH)], sv, sa)
        pltpu.async_copy(dst_hbm.at[pl.ds(dst_off(j), CH)], dv, sa)

    def wait_idx(b):
        sv, dv, _, sa, _ = bufs[b]
        pltpu.make_async_copy(src_hbm.at[pl.ds(0, CH)], sv, sa).wait()
        pltpu.make_async_copy(dst_hbm.at[pl.ds(0, CH)], dv, sa).wait()

    def fire_gather(b):
        sv, _, rv, _, sg = bufs[b]
        pltpu.async_copy(table.at[sv], rv, sg)

    def wait_gather(b):
        sv, _, rv, _, sg = bufs[b]
        pltpu.make_async_copy(table.at[sv], rv, sg).wait()

    def scatter(b):
        _, dv, rv, _, _ = bufs[b]
        pltpu.sync_copy(rv, accum.at[dv], add=True)

    n_even = n_chunks - (n_chunks % 2)
    fire_idx(0, 0)
    fire_idx(1, 1)
    wait_idx(0)
    fire_gather(0)
    last_t = n_even // 2 - 1

    def step(t, carry):
        wait_idx(1)
        fire_gather(1)
        wait_gather(0)
        scatter(0)

        @pl.when(t < last_t)
        def _():
            fire_idx(2 * t + 2, 0)
            wait_idx(0)
            fire_gather(0)

        wait_gather(1)
        scatter(1)

        @pl.when(t < last_t)
        def _():
            fire_idx(2 * t + 3, 1)

        return carry

    lax.fori_loop(0, n_even // 2, step, 0)
    if n_chunks % 2:
        fire_idx(n_chunks - 1, 0)
        wait_idx(0)
        fire_gather(0)
        wait_gather(0)
        scatter(0)


def _segsum_coarse_body(amsg, srcc, dstc, z, out, accum,
                        src_v0, dst_v0, rows_v0, src_v1, dst_v1, rows_v1,
                        semA0, semG0, semA1, semG1):
    c = lax.axis_index("c")
    s = lax.axis_index("s")
    w = c * 16 + s
    pltpu.sync_copy(z.at[pl.ds(0, 640)], accum.at[pl.ds(s * 640, 640)])
    plsc.subcore_barrier()
    base = w * (_CC_PER_TILE * CH)
    bufs = ((src_v0, dst_v0, rows_v0, semA0, semG0),
            (src_v1, dst_v1, rows_v1, semA1, semG1))
    _edge_pipeline(_CC_PER_TILE, lambda j: base + j * CH,
                   lambda j: base + j * CH, amsg, srcc, dstc, accum, bufs)

    @pl.when(s == 0)
    def _():
        off = _CC_BASE + c * CH
        pltpu.sync_copy(srcc.at[pl.ds(off, CH)], src_v0)
        pltpu.async_copy(amsg.at[src_v0], rows_v0, semG0).wait()
        pltpu.sync_copy(dstc.at[pl.ds(off, CH)], dst_v0)
        pltpu.sync_copy(rows_v0, accum.at[dst_v0], add=True)

    plsc.subcore_barrier()
    pltpu.sync_copy(accum.at[pl.ds(s * 640, 640)],
                    out.at[c, pl.ds(s * 640, 640)])


ICH = 80
_IQ = 1920 // ICH


def _interp_gather_body(ptab, idx3, gout,
                        idx_v0, rows_v0, idx_v1, rows_v1,
                        semA0, semG0, semA1, semG1):
    c = lax.axis_index("c")
    s = lax.axis_index("s")
    w = c * 16 + s
    bufs = ((idx_v0, rows_v0, semA0, semG0), (idx_v1, rows_v1, semA1, semG1))

    ipk = 640 // ICH

    def korow(q):
        k = q // ipk
        return k, w * 640 + (q - k * ipk) * ICH

    def fire_idx(q, b):
        iv, _, sa, _ = bufs[b]
        k, row = korow(q)
        pltpu.async_copy(idx3.at[pl.ds(k * NFP + row, ICH)], iv, sa)

    def wait_idx(b):
        iv, _, sa, _ = bufs[b]
        pltpu.make_async_copy(idx3.at[pl.ds(0, ICH)], iv, sa).wait()

    def fire_gather(b):
        iv, rv, _, sg = bufs[b]
        pltpu.async_copy(ptab.at[iv], rv, sg)

    def wait_gather(b):
        iv, rv, _, sg = bufs[b]
        pltpu.make_async_copy(ptab.at[iv], rv, sg).wait()

    def writeback(q, b):
        _, rv, _, _ = bufs[b]
        k, row = korow(q)
        pltpu.sync_copy(rv, gout.at[k, pl.ds(row, ICH)])

    fire_idx(0, 0)
    fire_idx(1, 1)
    wait_idx(0)
    fire_gather(0)
    last_t = _IQ // 2 - 1

    def step(t, carry):
        wait_idx(1)
        fire_gather(1)
        wait_gather(0)
        writeback(2 * t, 0)

        @pl.when(t < last_t)
        def _():
            fire_idx(2 * t + 2, 0)
            wait_idx(0)
            fire_gather(0)

        wait_gather(1)
        writeback(2 * t + 1, 1)

        @pl.when(t < last_t)
        def _():
            fire_idx(2 * t + 3, 1)

        return carry

    lax.fori_loop(0, _IQ // 2, step, 0)


def _segsum_fine_body(pm, src4, dstf, z, out, accum,
                      src_v0, dst_v0, rows_v0, src_v1, dst_v1, rows_v1,
                      semA0, semG0, semA1, semG1):
    c = lax.axis_index("c")
    s = lax.axis_index("s")
    bufs = ((src_v0, dst_v0, rows_v0, semA0, semG0),
            (src_v1, dst_v1, rows_v1, semA1, semG1))
    base = s * (_CF_PER_TILE * CH)
    for gi in range(2):
        g = c * 2 + gi
        goff = g * E_F
        pltpu.sync_copy(z, accum.at[pl.ds(s * 1280, 1280)])
        plsc.subcore_barrier()
        _edge_pipeline(_CF_PER_TILE,
                       lambda j, goff=goff: goff + base + j * CH,
                       lambda j: base + j * CH, pm, src4, dstf, accum, bufs)

        @pl.when(s < 4)
        def _(goff=goff):
            off = _CF_BASE + s * CH
            pltpu.sync_copy(src4.at[pl.ds(goff + off, CH)], src_v0)
            pltpu.async_copy(pm.at[src_v0], rows_v0, semG0).wait()
            pltpu.sync_copy(dstf.at[pl.ds(off, CH)], dst_v0)
            pltpu.sync_copy(rows_v0, accum.at[dst_v0], add=True)

        plsc.subcore_barrier()
        pltpu.sync_copy(accum.at[pl.ds(s * 1280, 1280)],
                        out.at[g, pl.ds(s * 1280, 1280)])
        plsc.subcore_barrier()


@functools.lru_cache(maxsize=1)
def _sc_kernels():
    mesh = plsc.VectorSubcoreMesh(core_axis_name="c", subcore_axis_name="s")
    params = pltpu.CompilerParams(use_tc_tiling_on_sc=False)
    edge_scratch = [
        pltpu.VMEM((CH,), I32),
        pltpu.VMEM((CH,), I32),
        pltpu.VMEM((CH, C_MID), F32),
        pltpu.VMEM((CH,), I32),
        pltpu.VMEM((CH,), I32),
        pltpu.VMEM((CH, C_MID), F32),
        pltpu.SemaphoreType.DMA,
        pltpu.SemaphoreType.DMA,
        pltpu.SemaphoreType.DMA,
        pltpu.SemaphoreType.DMA,
    ]
    segsum_coarse = pl.kernel(
        _segsum_coarse_body,
        out_type=jax.ShapeDtypeStruct((2, NCP, C_MID), F32),
        mesh=mesh,
        scratch_types=[pltpu.VMEM_SHARED((NCP, C_MID), F32)] + edge_scratch,
        compiler_params=params,
    )
    interp_gather = pl.kernel(
        _interp_gather_body,
        out_type=jax.ShapeDtypeStruct((3, NFP, 4 * C_OUT), F32),
        mesh=mesh,
        scratch_types=[
            pltpu.VMEM((ICH,), I32),
            pltpu.VMEM((ICH, 4 * C_OUT), F32),
            pltpu.VMEM((ICH,), I32),
            pltpu.VMEM((ICH, 4 * C_OUT), F32),
            pltpu.SemaphoreType.DMA,
            pltpu.SemaphoreType.DMA,
            pltpu.SemaphoreType.DMA,
            pltpu.SemaphoreType.DMA,
        ],
        compiler_params=params,
    )
    segsum_fine = pl.kernel(
        _segsum_fine_body,
        out_type=jax.ShapeDtypeStruct((4, NFP, C_MID), F32),
        mesh=mesh,
        scratch_types=[pltpu.VMEM_SHARED((NFP, C_MID), F32)] + edge_scratch,
        compiler_params=params,
    )
    return segsum_coarse, interp_gather, segsum_fine


def kernel(x, mesh_pos, m_pos_new, W_self1, W_msg1, b1, W_self2, W_msg2, b2,
           Ws_self, Ws_msg, bs, gamma, beta, edge_index, edge_index_fine):
    psn = jnp.pad(mesh_pos.astype(F32), ((0, NKP - N_C), (0, 5)),
                  constant_values=1e6)
    pd_blocks = (jnp.pad(m_pos_new.astype(F32), ((0, NFP - N_F), (0, 5)))
                 .reshape(KNBLK, KBLK, 8).transpose(0, 2, 1))
    src_c = edge_index[0].astype(I32)
    dst_c = edge_index[1].astype(I32)
    src_f = edge_index_fine[0].astype(I32)
    dst_f = edge_index_fine[1].astype(I32)
    src4 = (src_f[None, :]
            + (jnp.arange(4, dtype=I32) * NFP)[:, None]).reshape(4 * E_F)
    z = jnp.zeros((1280, C_MID), F32)
    _segsum_coarse, _interp_gather, _segsum_fine = _sc_kernels()

    idxo, wno = _knn(psn, pd_blocks)
    idx3 = idxo.transpose(1, 0, 2).reshape(3 * NFP)
    wn8 = jnp.pad(wno.transpose(0, 2, 1).reshape(NFP, 3), ((0, 0), (0, 5)))

    a_self, a_msg = _m1(x, W_self1, W_msg1)
    agg_c = _segsum_coarse(a_msg, src_c, dst_c, z)
    p = _p_kernel(a_self, agg_c, b1.reshape(1, C_MID), x,
                  W_self2, W_msg2, Ws_self, Ws_msg)

    gtab = _interp_gather(p, idx3)
    pself_f, pmsg = _w_kernel(gtab, wn8)

    agg_f = _segsum_fine(pmsg.reshape(4 * NFP, C_MID), src4, dst_f, z)

    o, stats = _f1(pself_f, agg_f, b2.reshape(1, C_OUT), bs.reshape(1, C_OUT))
    out = _f3(o, stats, gamma.reshape(1, C_OUT), beta.reshape(1, C_OUT))
    return out[:N_F]


# speedup vs baseline: 1.1008x; 1.0375x over previous
import functools

import jax
import jax.numpy as jnp
from jax import lax
from jax.experimental import pallas as pl
from jax.experimental.pallas import tpu as pltpu
from jax.experimental.pallas import tpu_sc as plsc

F32 = jnp.float32
I32 = jnp.int32

N_C = 10000
N_F = 20000
NFP = 20480
NCP = 10240
C_IN = 128
C_MID = 64
C_OUT = 128
E_C = 160000
E_F = 320000

BLK = 512
NBLK = NFP // BLK

SELU_ALPHA = 1.6732632423543772
SELU_SCALE = 1.0507009873554805


def _selu(v):
    return SELU_SCALE * jnp.where(v > 0, v, SELU_ALPHA * (jnp.exp(v) - 1.0))


KBLK = 640
KNBLK = NFP // KBLK
NKP = 10112
_BIG_I = NCP
_INF = 3e38


def _knn_body(ps_ref, pd_ref, idx_ref, wn_ref, d2_ref):
    acc = None
    for d in range(3):
        ps_d = ps_ref[:, d:d + 1]
        pd_d = pd_ref[0, d:d + 1, :]
        t = ps_d - pd_d
        t = t * t
        acc = t if acc is None else acc + t
    d2_ref[...] = acc
    iota = lax.broadcasted_iota(I32, (NKP, KBLK), 0)
    t = d2_ref[...]
    m1 = jnp.min(t, axis=0, keepdims=True)
    a1 = jnp.min(jnp.where(t == m1, iota, _BIG_I), axis=0, keepdims=True)
    not1 = iota != a1
    m2 = jnp.min(jnp.where(not1, t, _INF), axis=0, keepdims=True)
    a2 = jnp.min(jnp.where((t == m2) & not1, iota, _BIG_I), axis=0,
                 keepdims=True)
    not12 = not1 & (iota != a2)
    m3 = jnp.min(jnp.where(not12, t, _INF), axis=0, keepdims=True)
    a3 = jnp.min(jnp.where((t == m3) & not12, iota, _BIG_I), axis=0,
                 keepdims=True)
    w1 = 1.0 / (jnp.maximum(m1, 0.0) + 1e-8)
    w2 = 1.0 / (jnp.maximum(m2, 0.0) + 1e-8)
    w3 = 1.0 / (jnp.maximum(m3, 0.0) + 1e-8)
    s = w1 + w2 + w3
    for k, (a, w) in enumerate(((a1, w1), (a2, w2), (a3, w3))):
        idx_ref[0, k:k + 1, :] = a
        wn_ref[0, k:k + 1, :] = w / s


def _knn(ps_pad, pd_blocks):
    return pl.pallas_call(
        _knn_body,
        grid=(KNBLK,),
        in_specs=[
            pl.BlockSpec((NKP, 8), lambda i: (0, 0)),
            pl.BlockSpec((1, 8, KBLK), lambda i: (i, 0, 0)),
        ],
        out_specs=[
            pl.BlockSpec((1, 3, KBLK), lambda i: (i, 0, 0)),
            pl.BlockSpec((1, 3, KBLK), lambda i: (i, 0, 0)),
        ],
        out_shape=[
            jax.ShapeDtypeStruct((KNBLK, 3, KBLK), I32),
            jax.ShapeDtypeStruct((KNBLK, 3, KBLK), F32),
        ],
        scratch_shapes=[pltpu.VMEM((NKP, KBLK), F32)],
    )(ps_pad, pd_blocks)


M1_BLK = 1000


def _m1_body(x_ref, ws_ref, wm_ref, as_ref, am_ref):
    xb = x_ref[...]
    as_ref[...] = jnp.dot(xb, ws_ref[...], preferred_element_type=F32)
    am_ref[...] = jnp.dot(xb, wm_ref[...], preferred_element_type=F32)


def _m1(x, ws, wm):
    return pl.pallas_call(
        _m1_body,
        grid=(N_C // M1_BLK,),
        in_specs=[
            pl.BlockSpec((M1_BLK, C_IN), lambda i: (i, 0)),
            pl.BlockSpec((C_IN, C_MID), lambda i: (0, 0)),
            pl.BlockSpec((C_IN, C_MID), lambda i: (0, 0)),
        ],
        out_specs=[
            pl.BlockSpec((M1_BLK, C_MID), lambda i: (i, 0)),
            pl.BlockSpec((M1_BLK, C_MID), lambda i: (i, 0)),
        ],
        out_shape=[jax.ShapeDtypeStruct((N_C, C_MID), F32)] * 2,
    )(x, ws, wm)


def _p_body(as_ref, agg_ref, b1_ref, x_ref, w2s_ref, w2m_ref, wss_ref,
            wsm_ref, p_ref):
    h = _selu(as_ref[...] + agg_ref[0] + agg_ref[1] + b1_ref[...])
    xb = x_ref[...]
    p_ref[...] = jnp.concatenate(
        [
            jnp.dot(h, w2s_ref[...], preferred_element_type=F32),
            jnp.dot(xb, wss_ref[...], preferred_element_type=F32),
            jnp.dot(h, w2m_ref[...], preferred_element_type=F32),
            jnp.dot(xb, wsm_ref[...], preferred_element_type=F32),
        ],
        axis=1,
    )


def _p_kernel(a_self, agg_c, b1, x, w2s, w2m, wss, wsm):
    return pl.pallas_call(
        _p_body,
        grid=(N_C // M1_BLK,),
        in_specs=[
            pl.BlockSpec((M1_BLK, C_MID), lambda i: (i, 0)),
            pl.BlockSpec((2, M1_BLK, C_MID), lambda i: (0, i, 0)),
            pl.BlockSpec((1, C_MID), lambda i: (0, 0)),
            pl.BlockSpec((M1_BLK, C_IN), lambda i: (i, 0)),
            pl.BlockSpec((C_MID, C_OUT), lambda i: (0, 0)),
            pl.BlockSpec((C_MID, C_OUT), lambda i: (0, 0)),
            pl.BlockSpec((C_IN, C_OUT), lambda i: (0, 0)),
            pl.BlockSpec((C_IN, C_OUT), lambda i: (0, 0)),
        ],
        out_specs=pl.BlockSpec((M1_BLK, 4 * C_OUT), lambda i: (i, 0)),
        out_shape=jax.ShapeDtypeStruct((N_C, 4 * C_OUT), F32),
    )(a_self, agg_c, b1, x, w2s, w2m, wss, wsm)


def _w_body(g_ref, wn_ref, pself_ref, pmsg_ref):
    p = None
    for k in range(3):
        wk = wn_ref[:, k:k + 1]
        t = g_ref[k] * wk
        p = t if p is None else p + t
    pself_ref[...] = p[:, :2 * C_OUT]
    for g in range(4):
        pmsg_ref[g] = p[:, 2 * C_OUT + C_MID * g: 2 * C_OUT + C_MID * (g + 1)]


def _w_kernel(gtab, wn8):
    return pl.pallas_call(
        _w_body,
        grid=(NBLK,),
        in_specs=[
            pl.BlockSpec((3, BLK, 4 * C_OUT), lambda i: (0, i, 0)),
            pl.BlockSpec((BLK, 8), lambda i: (i, 0)),
        ],
        out_specs=[
            pl.BlockSpec((BLK, 2 * C_OUT), lambda i: (i, 0)),
            pl.BlockSpec((4, BLK, C_MID), lambda i: (0, i, 0)),
        ],
        out_shape=[
            jax.ShapeDtypeStruct((NFP, 2 * C_OUT), F32),
            jax.ShapeDtypeStruct((4, NFP, C_MID), F32),
        ],
    )(gtab, wn8)


def _f1_body(ps_ref, ag_ref, b2_ref, bs_ref, o_ref, st_ref):
    i = pl.program_id(0)
    ms = ps_ref[:, :C_OUT]
    ss = ps_ref[:, C_OUT:]
    am = jnp.concatenate([ag_ref[0], ag_ref[1]], axis=1)
    ak = jnp.concatenate([ag_ref[2], ag_ref[3]], axis=1)
    o = _selu(ms + am + b2_ref[...]) + _selu(ss + ak + bs_ref[...])
    o_ref[...] = o
    rows = i * BLK + lax.broadcasted_iota(I32, (BLK, 1), 0)
    ov = jnp.where(rows < N_F, o, 0.0)
    s1 = jnp.sum(ov, axis=0, keepdims=True)
    s2 = jnp.sum(ov * ov, axis=0, keepdims=True)

    @pl.when(i == 0)
    def _():
        st_ref[...] = jnp.zeros((8, C_OUT), F32)

    st_ref[0:1, :] += s1
    st_ref[1:2, :] += s2


def _f1(pself_f, agg_f, b2, bs):
    return pl.pallas_call(
        _f1_body,
        grid=(NBLK,),
        in_specs=[
            pl.BlockSpec((BLK, 2 * C_OUT), lambda i: (i, 0)),
            pl.BlockSpec((4, BLK, C_MID), lambda i: (0, i, 0)),
            pl.BlockSpec((1, C_OUT), lambda i: (0, 0)),
            pl.BlockSpec((1, C_OUT), lambda i: (0, 0)),
        ],
        out_specs=[
            pl.BlockSpec((BLK, C_OUT), lambda i: (i, 0)),
            pl.BlockSpec((8, C_OUT), lambda i: (0, 0)),
        ],
        out_shape=[
            jax.ShapeDtypeStruct((NFP, C_OUT), F32),
            jax.ShapeDtypeStruct((8, C_OUT), F32),
        ],
    )(pself_f, agg_f, b2, bs)


def _f3_body(o_ref, st_ref, g_ref, b_ref, out_ref):
    mean = st_ref[0:1, :] / N_F
    ex2 = st_ref[1:2, :] / N_F
    var = ex2 - mean * mean
    inv = lax.rsqrt(var + 1e-5)
    out_ref[...] = _selu((o_ref[...] - mean) * inv * g_ref[...] + b_ref[...])


def _f3(o, stats, gamma, beta):
    return pl.pallas_call(
        _f3_body,
        grid=(NBLK,),
        in_specs=[
            pl.BlockSpec((BLK, C_OUT), lambda i: (i, 0)),
            pl.BlockSpec((8, C_OUT), lambda i: (0, 0)),
            pl.BlockSpec((1, C_OUT), lambda i: (0, 0)),
            pl.BlockSpec((1, C_OUT), lambda i: (0, 0)),
        ],
        out_specs=pl.BlockSpec((BLK, C_OUT), lambda i: (i, 0)),
        out_shape=jax.ShapeDtypeStruct((NFP, C_OUT), F32),
    )(o, stats, gamma, beta)


CH = 128

_CC_PER_TILE = 39
_CC_BASE = 32 * _CC_PER_TILE * CH
_CF_PER_TILE = 156
_CF_BASE = 16 * _CF_PER_TILE * CH


def _edge_pipeline(n_chunks, src_off, dst_off, table, src_hbm, dst_hbm,
                   accum, bufs):
    def fire_idx(j, b):
        sv, dv, _, sa, _ = bufs[b]
        pltpu.async_copy(src_hbm.at[pl.ds(src_off(j), CH)], sv, sa)
        pltpu.async_copy(dst_hbm.at[pl.ds(dst_off(j), CH)], dv, sa)

    def wait_idx(b):
        sv, dv, _, sa, _ = bufs[b]
        pltpu.make_async_copy(src_hbm.at[pl.ds(0, CH)], sv, sa).wait()
        pltpu.make_async_copy(dst_hbm.at[pl.ds(0, CH)], dv, sa).wait()

    def fire_gather(b):
        sv, _, rv, _, sg = bufs[b]
        pltpu.async_copy(table.at[sv], rv, sg)

    def wait_gather(b):
        sv, _, rv, _, sg = bufs[b]
        pltpu.make_async_copy(table.at[sv], rv, sg).wait()

    def scatter(b):
        _, dv, rv, _, _ = bufs[b]
        pltpu.sync_copy(rv, accum.at[dv], add=True)

    n_even = n_chunks - (n_chunks % 2)
    fire_idx(0, 0)
    fire_idx(1, 1)
    wait_idx(0)
    fire_gather(0)
    last_t = n_even // 2 - 1

    def step(t, carry):
        wait_idx(1)
        fire_gather(1)
        wait_gather(0)
        scatter(0)

        @pl.when(t < last_t)
        def _():
            fire_idx(2 * t + 2, 0)
            wait_idx(0)
            fire_gather(0)

        wait_gather(1)
        scatter(1)

        @pl.when(t < last_t)
        def _():
            fire_idx(2 * t + 3, 1)

        return carry

    lax.fori_loop(0, n_even // 2, step, 0)
    if n_chunks % 2:
        fire_idx(n_chunks - 1, 0)
        wait_idx(0)
        fire_gather(0)
        wait_gather(0)
        scatter(0)


def _segsum_coarse_body(amsg, srcc, dstc, z, out, accum,
                        src_v0, dst_v0, rows_v0, src_v1, dst_v1, rows_v1,
                        semA0, semG0, semA1, semG1):
    c = lax.axis_index("c")
    s = lax.axis_index("s")
    w = c * 16 + s
    pltpu.sync_copy(z.at[pl.ds(0, 640)], accum.at[pl.ds(s * 640, 640)])
    plsc.subcore_barrier()
    base = w * (_CC_PER_TILE * CH)
    bufs = ((src_v0, dst_v0, rows_v0, semA0, semG0),
            (src_v1, dst_v1, rows_v1, semA1, semG1))
    _edge_pipeline(_CC_PER_TILE, lambda j: base + j * CH,
                   lambda j: base + j * CH, amsg, srcc, dstc, accum, bufs)

    @pl.when(s == 0)
    def _():
        off = _CC_BASE + c * CH
        pltpu.sync_copy(srcc.at[pl.ds(off, CH)], src_v0)
        pltpu.async_copy(amsg.at[src_v0], rows_v0, semG0).wait()
        pltpu.sync_copy(dstc.at[pl.ds(off, CH)], dst_v0)
        pltpu.sync_copy(rows_v0, accum.at[dst_v0], add=True)

    plsc.subcore_barrier()
    pltpu.sync_copy(accum.at[pl.ds(s * 640, 640)],
                    out.at[c, pl.ds(s * 640, 640)])


ICH = 80
_IQ = 1920 // ICH


def _interp_gather_body(ptab, idx3, gout,
                        idx_v0, rows_v0, idx_v1, rows_v1,
                        semA0, semG0, semA1, semG1):
    c = lax.axis_index("c")
    s = lax.axis_index("s")
    w = c * 16 + s
    bufs = ((idx_v0, rows_v0, semA0, semG0), (idx_v1, rows_v1, semA1, semG1))

    ipk = 640 // ICH

    def korow(q):
        k = q // ipk
        return k, w * 640 + (q - k * ipk) * ICH

    def fire_idx(q, b):
        iv, _, sa, _ = bufs[b]
        k, row = korow(q)
        pltpu.async_copy(idx3.at[pl.ds(k * NFP + row, ICH)], iv, sa)

    def wait_idx(b):
        iv, _, sa, _ = bufs[b]
        pltpu.make_async_copy(idx3.at[pl.ds(0, ICH)], iv, sa).wait()

    def fire_gather(b):
        iv, rv, _, sg = bufs[b]
        pltpu.async_copy(ptab.at[iv], rv, sg)

    def wait_gather(b):
        iv, rv, _, sg = bufs[b]
        pltpu.make_async_copy(ptab.at[iv], rv, sg).wait()

    def writeback(q, b):
        _, rv, _, _ = bufs[b]
        k, row = korow(q)
        pltpu.sync_copy(rv, gout.at[k, pl.ds(row, ICH)])

    fire_idx(0, 0)
    fire_idx(1, 1)
    wait_idx(0)
    fire_gather(0)
    last_t = _IQ // 2 - 1

    def step(t, carry):
        wait_idx(1)
        fire_gather(1)
        wait_gather(0)
        writeback(2 * t, 0)

        @pl.when(t < last_t)
        def _():
            fire_idx(2 * t + 2, 0)
            wait_idx(0)
            fire_gather(0)

        wait_gather(1)
        writeback(2 * t + 1, 1)

        @pl.when(t < last_t)
        def _():
            fire_idx(2 * t + 3, 1)

        return carry

    lax.fori_loop(0, _IQ // 2, step, 0)


def _edge_pipeline4(n_chunks, src_off, dst_off, table, src_hbm, dst_hbm,
                    accum, islots, rbufs):
    nq = n_chunks // 4

    def fire_idx(j, sl):
        sv, dv, sa = islots[sl]
        pltpu.async_copy(src_hbm.at[pl.ds(src_off(j), CH)], sv, sa)
        pltpu.async_copy(dst_hbm.at[pl.ds(dst_off(j), CH)], dv, sa)

    def wait_idx(sl):
        sv, dv, sa = islots[sl]
        pltpu.make_async_copy(src_hbm.at[pl.ds(0, CH)], sv, sa).wait()
        pltpu.make_async_copy(dst_hbm.at[pl.ds(0, CH)], dv, sa).wait()

    def fire_gather(rb, sl):
        rv, sg = rbufs[rb]
        pltpu.async_copy(table.at[islots[sl][0]], rv, sg)

    def wait_gather(rb, sl):
        rv, sg = rbufs[rb]
        pltpu.make_async_copy(table.at[islots[sl][0]], rv, sg).wait()

    def scatter(rb, sl):
        pltpu.sync_copy(rbufs[rb][0], accum.at[islots[sl][1]], add=True)

    for sl in range(4):
        fire_idx(sl, sl)
    wait_idx(0)
    fire_gather(0, 0)

    def step(q, carry):
        c0 = 4 * q
        more = q < nq - 1
        wait_idx(1)
        fire_gather(1, 1)
        wait_gather(0, 0)
        scatter(0, 0)

        @pl.when(more)
        def _():
            fire_idx(c0 + 4, 0)

        wait_idx(2)
        fire_gather(0, 2)
        wait_gather(1, 1)
        scatter(1, 1)

        @pl.when(more)
        def _():
            fire_idx(c0 + 5, 1)

        wait_idx(3)
        fire_gather(1, 3)
        wait_gather(0, 2)
        scatter(0, 2)

        @pl.when(more)
        def _():
            fire_idx(c0 + 6, 2)
            wait_idx(0)
            fire_gather(0, 0)

        wait_gather(1, 3)
        scatter(1, 3)

        @pl.when(more)
        def _():
            fire_idx(c0 + 7, 3)

        return carry

    lax.fori_loop(0, nq, step, 0)


def _segsum_fine_body(pm, src4, dstf, z, out, accum,
                      src_v0, dst_v0, rows_v0, src_v1, dst_v1, rows_v1,
                      src_v2, dst_v2, src_v3, dst_v3,
                      semA0, semG0, semA1, semG1, semA2, semA3):
    c = lax.axis_index("c")
    s = lax.axis_index("s")
    islots = ((src_v0, dst_v0, semA0), (src_v1, dst_v1, semA1),
              (src_v2, dst_v2, semA2), (src_v3, dst_v3, semA3))
    rbufs = ((rows_v0, semG0), (rows_v1, semG1))
    base = s * (_CF_PER_TILE * CH)
    for gi in range(2):
        g = c * 2 + gi
        goff = g * E_F
        pltpu.sync_copy(z, accum.at[pl.ds(s * 1280, 1280)])
        plsc.subcore_barrier()
        _edge_pipeline4(_CF_PER_TILE,
                        lambda j, goff=goff: goff + base + j * CH,
                        lambda j: base + j * CH, pm, src4, dstf, accum,
                        islots, rbufs)

        @pl.when(s < 4)
        def _(goff=goff):
            off = _CF_BASE + s * CH
            pltpu.sync_copy(src4.at[pl.ds(goff + off, CH)], src_v0)
            pltpu.async_copy(pm.at[src_v0], rows_v0, semG0).wait()
            pltpu.sync_copy(dstf.at[pl.ds(off, CH)], dst_v0)
            pltpu.sync_copy(rows_v0, accum.at[dst_v0], add=True)

        plsc.subcore_barrier()
        pltpu.sync_copy(accum.at[pl.ds(s * 1280, 1280)],
                        out.at[g, pl.ds(s * 1280, 1280)])
        plsc.subcore_barrier()


@functools.lru_cache(maxsize=1)
def _sc_kernels():
    mesh = plsc.VectorSubcoreMesh(core_axis_name="c", subcore_axis_name="s")
    params = pltpu.CompilerParams(use_tc_tiling_on_sc=False)
    edge_scratch = [
        pltpu.VMEM((CH,), I32),
        pltpu.VMEM((CH,), I32),
        pltpu.VMEM((CH, C_MID), F32),
        pltpu.VMEM((CH,), I32),
        pltpu.VMEM((CH,), I32),
        pltpu.VMEM((CH, C_MID), F32),
        pltpu.SemaphoreType.DMA,
        pltpu.SemaphoreType.DMA,
        pltpu.SemaphoreType.DMA,
        pltpu.SemaphoreType.DMA,
    ]
    segsum_coarse = pl.kernel(
        _segsum_coarse_body,
        out_type=jax.ShapeDtypeStruct((2, NCP, C_MID), F32),
        mesh=mesh,
        scratch_types=[pltpu.VMEM_SHARED((NCP, C_MID), F32)] + edge_scratch,
        compiler_params=params,
    )
    interp_gather = pl.kernel(
        _interp_gather_body,
        out_type=jax.ShapeDtypeStruct((3, NFP, 4 * C_OUT), F32),
        mesh=mesh,
        scratch_types=[
            pltpu.VMEM((ICH,), I32),
            pltpu.VMEM((ICH, 4 * C_OUT), F32),
            pltpu.VMEM((ICH,), I32),
            pltpu.VMEM((ICH, 4 * C_OUT), F32),
            pltpu.SemaphoreType.DMA,
            pltpu.SemaphoreType.DMA,
            pltpu.SemaphoreType.DMA,
            pltpu.SemaphoreType.DMA,
        ],
        compiler_params=params,
    )
    segsum_fine = pl.kernel(
        _segsum_fine_body,
        out_type=jax.ShapeDtypeStruct((4, NFP, C_MID), F32),
        mesh=mesh,
        scratch_types=[
            pltpu.VMEM_SHARED((NFP, C_MID), F32),
            pltpu.VMEM((CH,), I32),
            pltpu.VMEM((CH,), I32),
            pltpu.VMEM((CH, C_MID), F32),
            pltpu.VMEM((CH,), I32),
            pltpu.VMEM((CH,), I32),
            pltpu.VMEM((CH, C_MID), F32),
            pltpu.VMEM((CH,), I32),
            pltpu.VMEM((CH,), I32),
            pltpu.VMEM((CH,), I32),
            pltpu.VMEM((CH,), I32),
            pltpu.SemaphoreType.DMA,
            pltpu.SemaphoreType.DMA,
            pltpu.SemaphoreType.DMA,
            pltpu.SemaphoreType.DMA,
            pltpu.SemaphoreType.DMA,
            pltpu.SemaphoreType.DMA,
        ],
        compiler_params=params,
    )
    return segsum_coarse, interp_gather, segsum_fine


def kernel(x, mesh_pos, m_pos_new, W_self1, W_msg1, b1, W_self2, W_msg2, b2,
           Ws_self, Ws_msg, bs, gamma, beta, edge_index, edge_index_fine):
    psn = jnp.pad(mesh_pos.astype(F32), ((0, NKP - N_C), (0, 5)),
                  constant_values=1e6)
    pd_blocks = (jnp.pad(m_pos_new.astype(F32), ((0, NFP - N_F), (0, 5)))
                 .reshape(KNBLK, KBLK, 8).transpose(0, 2, 1))
    src_c = edge_index[0].astype(I32)
    dst_c = edge_index[1].astype(I32)
    src_f = edge_index_fine[0].astype(I32)
    dst_f = edge_index_fine[1].astype(I32)
    src4 = (src_f[None, :]
            + (jnp.arange(4, dtype=I32) * NFP)[:, None]).reshape(4 * E_F)
    z = jnp.zeros((1280, C_MID), F32)
    _segsum_coarse, _interp_gather, _segsum_fine = _sc_kernels()

    idxo, wno = _knn(psn, pd_blocks)
    idx3 = idxo.transpose(1, 0, 2).reshape(3 * NFP)
    wn8 = jnp.pad(wno.transpose(0, 2, 1).reshape(NFP, 3), ((0, 0), (0, 5)))

    a_self, a_msg = _m1(x, W_self1, W_msg1)
    agg_c = _segsum_coarse(a_msg, src_c, dst_c, z)
    p = _p_kernel(a_self, agg_c, b1.reshape(1, C_MID), x,
                  W_self2, W_msg2, Ws_self, Ws_msg)

    gtab = _interp_gather(p, idx3)
    pself_f, pmsg = _w_kernel(gtab, wn8)

    agg_f = _segsum_fine(pmsg.reshape(4 * NFP, C_MID), src4, dst_f, z)

    o, stats = _f1(pself_f, agg_f, b2.reshape(1, C_OUT), bs.reshape(1, C_OUT))
    out = _f3(o, stats, gamma.reshape(1, C_OUT), beta.reshape(1, C_OUT))
    return out[:N_F]
